# Initial kernel scaffold; baseline (speedup 1.0000x reference)
import jax
import jax.numpy as jnp
from jax.experimental import pallas as pl


def kernel(x, table):
    raise NotImplementedError("write your pallas kernel here")


# speedup vs baseline: 1.7953x; 1.7953x over previous
import functools

import jax
import jax.numpy as jnp
from jax import lax
from jax.experimental import pallas as pl
from jax.experimental.pallas import tpu as pltpu
from jax.experimental.pallas import tpu_sc as plsc

D_MODEL = 64
N_IDX = 16384 * 50

_info = plsc.get_sparse_core_info()
NC = _info.num_cores
NS = _info.num_subcores
NW = NC * NS
PER_W = N_IDX // NW
CHUNK = 512
N_CHUNKS = PER_W // CHUNK

_mesh = plsc.VectorSubcoreMesh(core_axis_name="c", subcore_axis_name="s")


@functools.partial(
    pl.kernel,
    mesh=_mesh,
    out_type=jax.ShapeDtypeStruct((N_IDX, D_MODEL), jnp.float32),
    scratch_types=[
        pltpu.VMEM((CHUNK,), jnp.int32),
        pltpu.VMEM((CHUNK, D_MODEL), jnp.float32),
        pltpu.SemaphoreType.DMA,
    ],
    compiler_params=pltpu.CompilerParams(use_tc_tiling_on_sc=False),
)
def _embed_gather(x_hbm, table_hbm, out_hbm, idx_v, rows_v, sem):
    wid = lax.axis_index("s") * NC + lax.axis_index("c")
    base = wid * PER_W

    def body(g, carry):
        off = base + g * CHUNK
        pltpu.sync_copy(x_hbm.at[pl.ds(off, CHUNK)], idx_v)
        pltpu.async_copy(table_hbm.at[idx_v], rows_v, sem).wait()
        pltpu.sync_copy(rows_v, out_hbm.at[pl.ds(off, CHUNK)])
        return carry

    lax.fori_loop(0, N_CHUNKS, body, 0)


def kernel(x, table):
    x_flat = x.reshape(-1).astype(jnp.int32)
    out = _embed_gather(x_flat, table)
    return out.reshape(x.shape + (table.shape[1],))


# speedup vs baseline: 1.8761x; 1.0450x over previous
import functools

import jax
import jax.numpy as jnp
from jax import lax
from jax.experimental import pallas as pl
from jax.experimental.pallas import tpu as pltpu
from jax.experimental.pallas import tpu_sc as plsc

D_MODEL = 64
N_IDX = 16384 * 50

_info = plsc.get_sparse_core_info()
NC = _info.num_cores
NS = _info.num_subcores
NW = NC * NS
PER_W = N_IDX // NW
CHUNK = 512
N_CHUNKS = PER_W // CHUNK
NBUF = 2
N_OUTER = N_CHUNKS // NBUF

_mesh = plsc.VectorSubcoreMesh(core_axis_name="c", subcore_axis_name="s")


@functools.partial(
    pl.kernel,
    mesh=_mesh,
    out_type=jax.ShapeDtypeStruct((N_IDX, D_MODEL), jnp.float32),
    scratch_types=[
        pltpu.VMEM((PER_W,), jnp.int32),
        pltpu.VMEM((CHUNK, D_MODEL), jnp.float32),
        pltpu.VMEM((CHUNK, D_MODEL), jnp.float32),
        pltpu.SemaphoreType.DMA,
        pltpu.SemaphoreType.DMA,
        pltpu.SemaphoreType.DMA,
        pltpu.SemaphoreType.DMA,
    ],
    compiler_params=pltpu.CompilerParams(use_tc_tiling_on_sc=False),
)
def _embed_gather(x_hbm, table_hbm, out_hbm, idx_v, rows0, rows1,
                  gs0, gs1, ss0, ss1):
    wid = lax.axis_index("s") * NC + lax.axis_index("c")
    base = wid * PER_W
    rows = [rows0, rows1]
    gsem = [gs0, gs1]
    ssem = [ss0, ss1]

    pltpu.sync_copy(x_hbm.at[pl.ds(base, PER_W)], idx_v)

    def idx_at(g):
        return idx_v.at[pl.ds(g * CHUNK, CHUNK)]

    def out_at(g):
        return out_hbm.at[pl.ds(base + g * CHUNK, CHUNK)]

    for b in range(NBUF):
        pltpu.async_copy(table_hbm.at[idx_at(b)], rows[b], gsem[b])

    def outer(i, carry):
        g0 = i * NBUF
        for b in range(NBUF):
            g = g0 + b
            pltpu.make_async_copy(table_hbm.at[idx_at(g)], rows[b],
                                  gsem[b]).wait()
            pltpu.async_copy(rows[b], out_at(g), ssem[b])
            pltpu.make_async_copy(rows[b], out_at(g), ssem[b]).wait()
            pltpu.async_copy(table_hbm.at[idx_at(g + NBUF)], rows[b],
                             gsem[b])
        return carry

    lax.fori_loop(0, N_OUTER - 1, outer, 0)

    g_last = (N_OUTER - 1) * NBUF
    for b in range(NBUF):
        g = g_last + b
        pltpu.make_async_copy(table_hbm.at[idx_at(g)], rows[b],
                              gsem[b]).wait()
        pltpu.async_copy(rows[b], out_at(g), ssem[b])
    for b in range(NBUF):
        g = g_last + b
        pltpu.make_async_copy(rows[b], out_at(g), ssem[b]).wait()


def kernel(x, table):
    x_flat = x.reshape(-1).astype(jnp.int32)
    out = _embed_gather(x_flat, table)
    return out.reshape(x.shape + (table.shape[1],))
